# baseline (speedup 1.0000x reference)
import jax
import jax.numpy as jnp
from jax import lax
from jax.experimental import pallas as pl
from jax.experimental.pallas import tpu as pltpu
from jax.experimental.pallas import tpu_sc as plsc

NC, NS = 2, 16
NW = NC * NS
B, S = 4096, 50
D = 64
RPW = B // NW


def _emb_body(idx_hbm, table_hbm, emb_hbm, idx_v, buf0, buf1, sem0, sem1):
    wid = lax.axis_index("s") * NC + lax.axis_index("c")
    base = wid * RPW

    pltpu.sync_copy(idx_hbm.at[pl.ds(base, RPW)], idx_v)

    bufs = (buf0, buf1)
    sems = (sem0, sem1)

    pltpu.async_copy(table_hbm.at[idx_v.at[0]], buf0, sem0).wait()

    def row_iter(j, carry):
        for p in range(2):
            @pl.when(j % 2 == p)
            def _do():
                g = pltpu.async_copy(table_hbm.at[idx_v.at[j]], bufs[p], sems[p])
                pltpu.sync_copy(bufs[1 - p], emb_hbm.at[base + j - 1])
                g.wait()
        return carry
    lax.fori_loop(1, RPW, row_iter, 0)

    pltpu.sync_copy(bufs[(RPW - 1) % 2], emb_hbm.at[base + RPW - 1])


def _mask_body(idx_ref, mask_ref):
    mask_ref[...] = jnp.where(idx_ref[...] > 0, 1.0, 0.0).astype(jnp.float32)


def kernel(string_lkup, table):
    mesh = plsc.VectorSubcoreMesh(core_axis_name="c", subcore_axis_name="s")
    emb = pl.kernel(
        _emb_body,
        out_type=jax.ShapeDtypeStruct((B, S, D), jnp.float32),
        mesh=mesh,
        compiler_params=pltpu.CompilerParams(use_tc_tiling_on_sc=False),
        scratch_types=[
            pltpu.VMEM((RPW, S), jnp.int32),
            pltpu.VMEM((S, D), jnp.float32),
            pltpu.VMEM((S, D), jnp.float32),
            pltpu.SemaphoreType.DMA,
            pltpu.SemaphoreType.DMA,
        ],
    )(string_lkup, table)

    mask = pl.pallas_call(
        _mask_body,
        out_shape=jax.ShapeDtypeStruct((B, S), jnp.float32),
    )(string_lkup)
    return emb, mask
